# Initial kernel scaffold; baseline (speedup 1.0000x reference)
import jax
import jax.numpy as jnp
from jax.experimental import pallas as pl


def kernel(x, atoms):
    raise NotImplementedError("write your pallas kernel here")


# speedup vs baseline: 10.3807x; 10.3807x over previous
import jax
import jax.numpy as jnp
from jax import lax
from jax.experimental import pallas as pl
from jax.experimental.pallas import tpu as pltpu
from jax.experimental.pallas import tpu_sc as plsc

_NS = 16384
_NA = 512
_AS = 512
_K = 128
_BLK = 128
_NB = _NS // _BLK
_NBLK = _NA * _NB
_CH = 2048
_NCH = _NS // _CH
_NEG = float("-inf")
_BIG = 2**30


def _k1_body(x_ref, m_ref):
    xr = x_ref[:].reshape(_NA, _CH // _BLK, _BLK)
    m = jnp.max(xr, axis=2)
    m_ref[:] = m.T


def _k1(x):
    batch = x.shape[0]
    return pl.pallas_call(
        _k1_body,
        grid=(batch, _NCH),
        in_specs=[pl.BlockSpec((None, _NA, _CH), lambda b, c: (b, 0, c))],
        out_specs=pl.BlockSpec((None, _CH // _BLK, _NA), lambda b, c: (b, c, 0)),
        out_shape=jax.ShapeDtypeStruct((batch, _NB, _NA), jnp.float32),
    )(x)


def _k2_body(m_ref, blk_ref, ms_ref):
    ms_ref[:] = m_ref[:]
    i_c = lax.broadcasted_iota(jnp.int32, (_NB, _NA), 0)
    i_r = lax.broadcasted_iota(jnp.int32, (_NB, _NA), 1)
    i_blk = i_r * _NB + i_c
    lane = lax.broadcasted_iota(jnp.int32, (1, _K), 1)

    def body(k, acc):
        m = ms_ref[:]
        vmax = jnp.max(m)
        fb = jnp.min(jnp.where(m == vmax, i_blk, _BIG))
        ms_ref[:] = jnp.where(i_blk == fb, _NEG, m)
        return jnp.where(lane == k, fb, acc)

    blk_ref[:] = lax.fori_loop(0, _K, body, jnp.zeros((1, _K), jnp.int32))


def _k2(m):
    batch = m.shape[0]
    return pl.pallas_call(
        _k2_body,
        grid=(batch,),
        in_specs=[pl.BlockSpec((None, _NB, _NA), lambda b: (b, 0, 0))],
        out_specs=pl.BlockSpec((None, 1, _K), lambda b: (b, 0, 0)),
        out_shape=jax.ShapeDtypeStruct((batch, 1, _K), jnp.int32),
        scratch_shapes=[pltpu.VMEM((_NB, _NA), jnp.float32)],
    )(m)


def _k3_body(blk_hbm, xrows_hbm, c_hbm, ids_v, idg_v, cand_v, sem):
    batch = blk_hbm.shape[0]
    c = lax.axis_index("c")
    s = lax.axis_index("s")
    wid = s * 2 + c
    b = lax.bitwise_and(wid, batch - 1)
    pltpu.sync_copy(blk_hbm.at[b], ids_v)
    for i in range(_K // 16):
        iv = ids_v[pl.ds(i * 16, 16)]
        idg_v[pl.ds(i * 16, 16)] = iv + b * _NBLK
    pltpu.make_async_copy(xrows_hbm.at[idg_v], cand_v, sem).wait()

    @pl.when(wid < batch)
    def _store():
        pltpu.sync_copy(cand_v, c_hbm.at[wid])


def _k3(blk, xrows):
    batch = blk.shape[0]
    f = pl.kernel(
        _k3_body,
        out_type=jax.ShapeDtypeStruct((batch, _K, _BLK), jnp.float32),
        mesh=plsc.VectorSubcoreMesh(core_axis_name="c", subcore_axis_name="s"),
        scratch_types=[
            pltpu.VMEM((_K,), jnp.int32),
            pltpu.VMEM((_K,), jnp.int32),
            pltpu.VMEM((_K, _BLK), jnp.float32),
            pltpu.SemaphoreType.DMA,
        ],
    )
    return f(blk, xrows)


def _k4_body(c_ref, blk_ref, vals_ref, gidx_ref, cs_ref):
    cs_ref[:] = c_ref[:]
    i_s = lax.broadcasted_iota(jnp.int32, (_K, _BLK), 0)
    i_j = lax.broadcasted_iota(jnp.int32, (_K, _BLK), 1)
    i_flat = i_s * _BLK + i_j
    lane = lax.broadcasted_iota(jnp.int32, (1, _K), 1)
    blkrow = blk_ref[:]

    def body(k, carry):
        vacc, iacc = carry
        m = cs_ref[:]
        vmax = jnp.max(m)
        flat = jnp.min(jnp.where(m == vmax, i_flat, _BIG))
        slot = flat // _BLK
        j = flat - slot * _BLK
        bid = jnp.min(jnp.where(lane == slot, blkrow, _BIG))
        gid = bid * _BLK + j
        cs_ref[:] = jnp.where(i_flat == flat, _NEG, m)
        vacc = jnp.where(lane == k, vmax, vacc)
        iacc = jnp.where(lane == k, gid, iacc)
        return (vacc, iacc)

    vacc, iacc = lax.fori_loop(
        0, _K, body,
        (jnp.zeros((1, _K), jnp.float32), jnp.zeros((1, _K), jnp.int32)))
    vals_ref[:] = vacc
    gidx_ref[:] = iacc


def _k4(cand, blk):
    batch = cand.shape[0]
    return pl.pallas_call(
        _k4_body,
        grid=(batch,),
        in_specs=[
            pl.BlockSpec((None, _K, _BLK), lambda b: (b, 0, 0)),
            pl.BlockSpec((None, 1, _K), lambda b: (b, 0, 0)),
        ],
        out_specs=[
            pl.BlockSpec((None, 1, _K), lambda b: (b, 0, 0)),
            pl.BlockSpec((None, 1, _K), lambda b: (b, 0, 0)),
        ],
        out_shape=[
            jax.ShapeDtypeStruct((batch, 1, _K), jnp.float32),
            jax.ShapeDtypeStruct((batch, 1, _K), jnp.int32),
        ],
        scratch_shapes=[pltpu.VMEM((_K, _BLK), jnp.float32)],
    )(cand, blk)


def _take16(v, idx):
    return lax.gather(
        v, idx[:, None],
        lax.GatherDimensionNumbers(
            offset_dims=(), collapsed_slice_dims=(0,), start_index_map=(0,)),
        (1,), mode=lax.GatherScatterMode.PROMISE_IN_BOUNDS)


def _k5_body(vals_hbm, idx_hbm, atoms_hbm, out_hbm,
             vals_v, idx_v, aid_v, rows_v, acc_v, sem):
    batch = vals_hbm.shape[0]
    pad = _NS + _AS
    c = lax.axis_index("c")
    s = lax.axis_index("s")
    wid = s * 2 + c
    b = lax.bitwise_and(wid, batch - 1)

    pltpu.sync_copy(vals_hbm.at[b], vals_v)
    pltpu.sync_copy(idx_hbm.at[b], idx_v)
    for i in range(_K // 16):
        iv = idx_v[pl.ds(i * 16, 16)]
        aid_v[pl.ds(i * 16, 16)] = lax.shift_right_logical(iv, 14)
    pltpu.make_async_copy(atoms_hbm.at[aid_v], rows_v, sem).wait()

    zeros16 = jnp.zeros((16,), jnp.float32)

    def zbody(t, carry):
        acc_v[pl.ds(t * 16, 16)] = zeros16
        return carry
    lax.fori_loop(0, pad // 16, zbody, 0)

    lane = lax.iota(jnp.int32, 16)
    zseg = jnp.zeros((16,), jnp.float32)
    nseg = _AS // 16
    for chunk in range(_K // 16):
        v16 = vals_v[pl.ds(chunk * 16, 16)]
        i16 = idx_v[pl.ds(chunk * 16, 16)]
        s16 = lax.bitwise_and(i16, _NS - 1)
        for t in range(16):
            v = v16[t]
            samp = s16[t]
            k = chunk * 16 + t
            q16 = lax.bitwise_and(samp, ~15)
            rem = lax.bitwise_and(samp, 15)
            rot = lax.bitwise_and(lane - rem, 15)
            inlo = lane < rem

            def jbody(j, prev, k=k, v=v, q16=q16, rot=rot, inlo=inlo):
                cur = jnp.where(
                    j < nseg,
                    rows_v[k, pl.ds(lax.min(j, nseg - 1) * 16, 16)],
                    zseg)
                g_prev = _take16(prev, rot)
                g_cur = _take16(cur, rot)
                seg = jnp.where(inlo, g_prev, g_cur)
                start = q16 + j * 16
                acc_v[pl.ds(start, 16)] = acc_v[pl.ds(start, 16)] + v * seg
                return cur
            lax.fori_loop(0, nseg + 1, jbody, zseg)

    @pl.when(wid < batch)
    def _store():
        pltpu.sync_copy(acc_v.at[pl.ds(0, _NS)], out_hbm.at[wid])


def _k5(vals, idx, atoms2d):
    batch = vals.shape[0]
    pad = _NS + _AS
    f = pl.kernel(
        _k5_body,
        out_type=jax.ShapeDtypeStruct((batch, _NS), jnp.float32),
        mesh=plsc.VectorSubcoreMesh(core_axis_name="c", subcore_axis_name="s"),
        scratch_types=[
            pltpu.VMEM((_K,), jnp.float32),
            pltpu.VMEM((_K,), jnp.int32),
            pltpu.VMEM((_K,), jnp.int32),
            pltpu.VMEM((_K, _AS), jnp.float32),
            pltpu.VMEM((pad,), jnp.float32),
            pltpu.SemaphoreType.DMA,
        ],
    )
    return f(vals, idx, atoms2d)


def kernel(x, atoms):
    batch = x.shape[0]
    m = _k1(x)
    blk = _k2(m)
    xrows = x.reshape(batch, _NA * _NB, _BLK)
    cand = jnp.take_along_axis(xrows, blk.reshape(batch, _K)[:, :, None], axis=1)
    vals, gidx = _k4(cand, blk)
    vals = vals.reshape(batch, _K)
    gidx = gidx.reshape(batch, _K)
    atom_index = gidx // _NS
    sample_index = gidx % _NS
    atoms_b = jnp.broadcast_to(atoms, (batch, _NA, _AS))
    gathered = jnp.take_along_axis(atoms_b, atom_index[:, :, None], axis=1)
    scaled = gathered * vals[:, :, None]
    pos = sample_index[:, :, None] + jnp.arange(_AS)[None, None, :]
    b_idx = jnp.broadcast_to(jnp.arange(batch)[:, None, None], pos.shape)
    out = jnp.zeros((batch, _NS + _AS), dtype=x.dtype)
    out = out.at[b_idx, pos].add(scaled)
    return out[:, None, :_NS]
